# Initial kernel scaffold; baseline (speedup 1.0000x reference)
import jax
import jax.numpy as jnp
from jax.experimental import pallas as pl


def kernel(nodes, neighbors, u2e_weight, att1_W, att1_b, att2_W, att2_b, att3_W, att3_b, lin1_W, lin1_b):
    raise NotImplementedError("write your pallas kernel here")


# speedup vs baseline: 1.8300x; 1.8300x over previous
import jax
import jax.numpy as jnp
from jax.experimental import pallas as pl
from jax.experimental.pallas import tpu as pltpu
from jax.experimental.pallas import tpu_sc as plsc

NUM_NODES = 16384
NUM_NEIGHBORS = 50
EMBED_DIM = 64

_GATHER_WINDOW = 256
_BLOCK_NODES = 256


def _sc_gather(table, idx):
    total = idx.shape[0]
    d = table.shape[1]
    mesh = plsc.VectorSubcoreMesh(core_axis_name="core", subcore_axis_name="subcore")
    idx2 = idx.reshape(1, total)

    @pl.kernel(
        out_type=jax.ShapeDtypeStruct((total, d), table.dtype),
        mesh=mesh,
    )
    def gather_kernel(x_hbm, i_hbm, o_hbm):
        def body(i_vmem, o_vmem):
            pltpu.sync_copy(x_hbm.at[i_vmem.at[0]], o_vmem)

        pltpu.emit_pipeline(
            body,
            grid=(total // _GATHER_WINDOW,),
            in_specs=[pl.BlockSpec((1, _GATHER_WINDOW), index_map=lambda i: (0, i))],
            out_specs=[pl.BlockSpec((_GATHER_WINDOW, d), index_map=lambda i: (i, 0))],
            core_axis_name=("core", "subcore"),
            dimension_semantics=(pltpu.PARALLEL,),
        )(i_hbm, o_hbm)

    return gather_kernel(table, idx2)


def _mlp_body(g_ref, par_ref, w1e_ref, w1u_ref, b1_ref, w2_ref, b2_ref,
              w3_ref, b3_ref, wls_ref, wln_ref, bl_ref, o_ref):
    K = NUM_NEIGHBORS
    B = o_ref.shape[0]
    D = o_ref.shape[1]
    g128 = g_ref[...]
    par = par_ref[...]
    g = jnp.where(par > 0.5, g128[:, :, D:], g128[:, :, :D])
    u = g[K]
    e3 = g[:K]

    h0 = jnp.dot(u, w1u_ref[...], preferred_element_type=jnp.float32) + b1_ref[...]
    e2 = e3.reshape(K * B, D)
    a = jnp.dot(e2, w1e_ref[...], preferred_element_type=jnp.float32)
    h1 = jax.nn.relu(a.reshape(K, B, D) + h0[None])

    h2 = jax.nn.relu(
        jnp.dot(h1.reshape(K * B, D), w2_ref[...],
                preferred_element_type=jnp.float32) + b2_ref[...]
    ).reshape(K, B, D)

    z = jnp.sum(h2 * w3_ref[...][None], axis=-1, keepdims=True) + b3_ref[0, 0]
    m = jnp.max(z, axis=0, keepdims=True)
    p = jnp.exp(z - m)
    att = p / jnp.sum(p, axis=0, keepdims=True)

    neigh = jnp.sum(e3 * att, axis=0)

    out = jax.nn.relu(
        jnp.dot(u, wls_ref[...], preferred_element_type=jnp.float32)
        + jnp.dot(neigh, wln_ref[...], preferred_element_type=jnp.float32)
        + bl_ref[...]
    )
    o_ref[...] = out


def _tc_mlp(g, par, w1e, w1u, b1, w2, b2, w3, b3, wls, wln, bl):
    n = g.shape[1]
    d = EMBED_DIM
    B = _BLOCK_NODES
    full = lambda shape: pl.BlockSpec(shape, lambda i: (0,) * len(shape))
    return pl.pallas_call(
        _mlp_body,
        grid=(n // B,),
        in_specs=[
            pl.BlockSpec((NUM_NEIGHBORS + 1, B, 2 * d), lambda i: (0, i, 0)),
            pl.BlockSpec((NUM_NEIGHBORS + 1, B, 1), lambda i: (0, i, 0)),
            full(w1e.shape), full(w1u.shape), full(b1.shape),
            full(w2.shape), full(b2.shape), full(w3.shape), full(b3.shape),
            full(wls.shape), full(wln.shape), full(bl.shape),
        ],
        out_specs=pl.BlockSpec((B, d), lambda i: (i, 0)),
        out_shape=jax.ShapeDtypeStruct((n, d), jnp.float32),
    )(g, par, w1e, w1u, b1, w2, b2, w3, b3, wls, wln, bl)


def kernel(nodes, neighbors, u2e_weight, att1_W, att1_b, att2_W, att2_b,
           att3_W, att3_b, lin1_W, lin1_b):
    D = EMBED_DIM
    idx = jnp.concatenate(
        [neighbors.T.reshape(-1), nodes]).astype(jnp.int32)
    table2 = u2e_weight.reshape(-1, 2 * D)
    gathered = _sc_gather(table2, idx >> 1)
    g = gathered.reshape(NUM_NEIGHBORS + 1, NUM_NODES, 2 * D)
    par = (idx & 1).astype(jnp.float32).reshape(
        NUM_NEIGHBORS + 1, NUM_NODES, 1)

    w1e = att1_W[:, :D].T
    w1u = att1_W[:, D:].T
    b1 = att1_b.reshape(1, D)
    w2 = att2_W.T
    b2 = att2_b.reshape(1, D)
    w3 = att3_W
    b3 = att3_b.reshape(1, 1)
    wls = lin1_W[:, :D].T
    wln = lin1_W[:, D:].T
    bl = lin1_b.reshape(1, D)

    return _tc_mlp(g, par, w1e, w1u, b1, w2, b2, w3, b3, wls, wln, bl)
